# baseline (speedup 1.0000x reference)
import functools

import jax
import jax.numpy as jnp
from jax import lax
from jax.experimental import pallas as pl
from jax.experimental.pallas import tpu as pltpu
from jax.experimental.pallas import tpu_sc as plsc

_INF = 3.0e38


def _sc_interp(idx3, w3, known_feats):
    B, C2, m = known_feats.shape
    n = idx3.shape[2]
    info = plsc.get_sparse_core_info()
    NC, NS, L = info.num_cores, info.num_subcores, info.num_lanes
    NW = NC * NS
    wpb = NW // B
    CC = C2 // wpb
    PC = 256
    QL = CC // L
    mesh = plsc.VectorSubcoreMesh(core_axis_name="c", subcore_axis_name="s")

    @functools.partial(
        pl.kernel, mesh=mesh,
        out_type=jax.ShapeDtypeStruct((NW * n * CC,), jnp.float32),
        compiler_params=pltpu.CompilerParams(needs_layout_passes=False),
        scratch_types=[
            pltpu.VMEM((3 * n,), jnp.int32),
            pltpu.VMEM((3 * n,), jnp.float32),
            pltpu.VMEM((m * CC,), jnp.float32),
            pltpu.VMEM((PC * CC,), jnp.float32),
            pltpu.VMEM((PC * CC,), jnp.float32),
            pltpu.SemaphoreType.DMA,
            pltpu.SemaphoreType.DMA,
        ],
    )
    def k(idx_hbm, w_hbm, kf_hbm, out_hbm, idxv, wv, tbl,
          obuf0, obuf1, sem0, sem1):
        wid = lax.axis_index("s") * NC + lax.axis_index("c")
        b = wid // wpb
        pltpu.sync_copy(idx_hbm.at[pl.ds(b * 3 * n, 3 * n)], idxv)
        pltpu.sync_copy(w_hbm.at[pl.ds(b * 3 * n, 3 * n)], wv)
        pltpu.sync_copy(kf_hbm.at[pl.ds(wid * m * CC, m * CC)], tbl)

        iota = lax.broadcasted_iota(jnp.int32, (L,), 0)
        qvecs = [iota + q * L for q in range(QL)]
        jvecs = [iota * 0 + j for j in range(L)]
        obufs = (obuf0, obuf1)
        sems = (sem0, sem1)

        def compute_chunk(pc, obuf):
            def grp_body(g, carry2):
                iv0 = idxv[pl.ds(0 * n + pc * PC + g * L, L)] * CC
                iv1 = idxv[pl.ds(1 * n + pc * PC + g * L, L)] * CC
                iv2 = idxv[pl.ds(2 * n + pc * PC + g * L, L)] * CC
                wv0 = wv[pl.ds(0 * n + pc * PC + g * L, L)]
                wv1 = wv[pl.ds(1 * n + pc * PC + g * L, L)]
                wv2 = wv[pl.ds(2 * n + pc * PC + g * L, L)]
                for j in range(L):
                    b0 = iv0[jvecs[j]]
                    b1 = iv1[jvecs[j]]
                    b2 = iv2[jvecs[j]]
                    w0 = wv0[jvecs[j]]
                    w1 = wv1[jvecs[j]]
                    w2 = wv2[jvecs[j]]
                    for q in range(QL):
                        v0 = plsc.load_gather(tbl, [b0 + qvecs[q]])
                        v1 = plsc.load_gather(tbl, [b1 + qvecs[q]])
                        v2 = plsc.load_gather(tbl, [b2 + qvecs[q]])
                        obuf[pl.ds((g * L + j) * CC + q * L, L)] = (
                            w0 * v0 + w1 * v1 + w2 * v2)
                return carry2

            lax.fori_loop(0, PC // L, grp_body, 0)

        def drain(par):
            pltpu.make_async_copy(
                obufs[par], out_hbm.at[pl.ds(wid * n * CC, PC * CC)],
                sems[par]).wait()

        def pair_body(t, carry):
            @pl.when(t > 0)
            def _():
                drain(0)

            compute_chunk(2 * t, obuf0)
            pltpu.async_copy(
                obuf0,
                out_hbm.at[pl.ds(wid * n * CC + 2 * t * PC * CC, PC * CC)],
                sem0)

            @pl.when(t > 0)
            def _():
                drain(1)

            compute_chunk(2 * t + 1, obuf1)
            pltpu.async_copy(
                obuf1,
                out_hbm.at[pl.ds(wid * n * CC + (2 * t + 1) * PC * CC,
                                 PC * CC)],
                sem1)
            return carry

        lax.fori_loop(0, n // PC // 2, pair_body, 0)
        drain(0)
        drain(1)

    kf_chunks = (known_feats.transpose(0, 2, 1)
                 .reshape(B, m, wpb, CC)
                 .transpose(0, 2, 1, 3)
                 .reshape(B * wpb * m * CC))
    out = k(idx3.reshape(B * 3 * n), w3.reshape(B * 3 * n), kf_chunks)
    return out.reshape(B, wpb, n, CC)


def _nn_kernel(unk_ref, kn_ref, idx_ref, w_ref, *, m):
    u = unk_ref[0]
    kn = kn_ref[0]
    d2 = None
    for c in range(3):
        diff = kn[:, c:c + 1] - u[c].reshape(1, -1)
        sq = diff * diff
        d2 = sq if d2 is None else d2 + sq
    iota = jax.lax.broadcasted_iota(jnp.int32, d2.shape, 0)
    key = jax.lax.bitcast_convert_type(d2, jnp.int32)
    key = jax.lax.bitwise_or(jax.lax.bitwise_and(key, ~(m - 1)), iota)
    idxs, dists = [], []
    for _ in range(3):
        mn = jnp.min(key, axis=0, keepdims=True)
        key = jnp.where(key == mn, jnp.int32(0x7FFFFFFF), key)
        idxs.append(jax.lax.bitwise_and(mn, m - 1))
        d2sel = jax.lax.bitcast_convert_type(
            jax.lax.bitwise_and(mn, ~(m - 1)), jnp.float32)
        dists.append(jnp.sqrt(d2sel))
    idx_ref[0] = jnp.concatenate(idxs, axis=0)
    dr = [1.0 / (dd + 1e-8) for dd in dists]
    norm = dr[0] + dr[1] + dr[2]
    w_ref[0] = jnp.concatenate([x / norm for x in dr], axis=0)


def _three_nn(unknown_t, known):
    B, _, n = unknown_t.shape
    m = known.shape[1]
    NB = min(n, 2048)
    grid = (B, n // NB)
    return pl.pallas_call(
        functools.partial(_nn_kernel, m=m),
        grid=grid,
        in_specs=[
            pl.BlockSpec((1, 3, NB), lambda b, i: (b, 0, i)),
            pl.BlockSpec((1, m, 3), lambda b, i: (b, 0, 0)),
        ],
        out_specs=[
            pl.BlockSpec((1, 3, NB), lambda b, i: (b, 0, i)),
            pl.BlockSpec((1, 3, NB), lambda b, i: (b, 0, i)),
        ],
        out_shape=[
            jax.ShapeDtypeStruct((B, 3, n), jnp.int32),
            jax.ShapeDtypeStruct((B, 3, n), jnp.float32),
        ],
    )(unknown_t, known)


def _mlp1b_kernel(uf_ref, W1b_ref, b1_ref, o_ref):
    o_ref[0] = jnp.dot(W1b_ref[...], uf_ref[0],
                       preferred_element_type=jnp.float32) + b1_ref[...]


def _mlp1b(unknow_feats, W1b, b1c):
    B, C1, n = unknow_feats.shape
    Co = W1b.shape[0]
    NB = min(n, 1024)
    grid = (B, n // NB)
    return pl.pallas_call(
        _mlp1b_kernel,
        grid=grid,
        in_specs=[
            pl.BlockSpec((1, C1, NB), lambda b, i: (b, 0, i)),
            pl.BlockSpec((Co, C1), lambda b, i: (0, 0)),
            pl.BlockSpec((Co, 1), lambda b, i: (0, 0)),
        ],
        out_specs=pl.BlockSpec((1, Co, NB), lambda b, i: (b, 0, i)),
        out_shape=jax.ShapeDtypeStruct((B, Co, n), jnp.float32),
    )(unknow_feats, W1b, b1c)


def _mlp1_kernel(in_ref, hb_ref, W1a_ref, h_ref, st_ref, *, wpb, CC):
    inr = in_ref[0]
    h = hb_ref[0]
    for j in range(wpb):
        h = h + jax.lax.dot_general(
            W1a_ref[:, j * CC:(j + 1) * CC], inr[j],
            (((1,), (1,)), ((), ())),
            preferred_element_type=jnp.float32)
    h_ref[0] = h

    @pl.when(jnp.logical_and(pl.program_id(0) == 0, pl.program_id(1) == 0))
    def _():
        st_ref[...] = jnp.zeros_like(st_ref)

    s0 = jnp.sum(h, axis=1, keepdims=True)
    s1 = jnp.sum(h * h, axis=1, keepdims=True)
    st_ref[...] += jnp.concatenate([s0, s1], axis=1)


def _mlp1(interp4, h1b, W1a):
    B, wpb, n, CC = interp4.shape
    C2 = wpb * CC
    Co = W1a.shape[0]
    NB = min(n, 512)
    grid = (B, n // NB)
    return pl.pallas_call(
        functools.partial(_mlp1_kernel, wpb=wpb, CC=CC),
        grid=grid,
        in_specs=[
            pl.BlockSpec((1, wpb, NB, CC), lambda b, i: (b, 0, i, 0)),
            pl.BlockSpec((1, Co, NB), lambda b, i: (b, 0, i)),
            pl.BlockSpec((Co, C2), lambda b, i: (0, 0)),
        ],
        out_specs=[
            pl.BlockSpec((1, Co, NB), lambda b, i: (b, 0, i)),
            pl.BlockSpec((Co, 2), lambda b, i: (0, 0)),
        ],
        out_shape=[
            jax.ShapeDtypeStruct((B, Co, n), jnp.float32),
            jax.ShapeDtypeStruct((Co, 2), jnp.float32),
        ],
    )(interp4, h1b, W1a)


def _mlp2_kernel(h_ref, st_ref, g_ref, be_ref, W_ref, b_ref,
                 o_ref, st2_ref, *, count):
    st = st_ref[...]
    mean = st[:, 0:1] * (1.0 / count)
    var = st[:, 1:2] * (1.0 / count) - mean * mean
    scale = g_ref[...] * jax.lax.rsqrt(var + 1e-5)
    x = jnp.maximum((h_ref[0] - mean) * scale + be_ref[...], 0.0)
    h2 = jnp.dot(W_ref[...], x, preferred_element_type=jnp.float32) + b_ref[...]
    o_ref[0] = h2

    @pl.when(jnp.logical_and(pl.program_id(0) == 0, pl.program_id(1) == 0))
    def _():
        st2_ref[...] = jnp.zeros_like(st2_ref)

    s0 = jnp.sum(h2, axis=1, keepdims=True)
    s1 = jnp.sum(h2 * h2, axis=1, keepdims=True)
    st2_ref[...] += jnp.concatenate([s0, s1], axis=1)


def _mlp2(h1, st1, g1c, be1c, W2, b2c, count):
    B, Ci, n = h1.shape
    Co = W2.shape[0]
    NB = min(n, 1024)
    grid = (B, n // NB)
    return pl.pallas_call(
        functools.partial(_mlp2_kernel, count=count),
        grid=grid,
        in_specs=[
            pl.BlockSpec((1, Ci, NB), lambda b, i: (b, 0, i)),
            pl.BlockSpec((Ci, 2), lambda b, i: (0, 0)),
            pl.BlockSpec((Ci, 1), lambda b, i: (0, 0)),
            pl.BlockSpec((Ci, 1), lambda b, i: (0, 0)),
            pl.BlockSpec((Co, Ci), lambda b, i: (0, 0)),
            pl.BlockSpec((Co, 1), lambda b, i: (0, 0)),
        ],
        out_specs=[
            pl.BlockSpec((1, Co, NB), lambda b, i: (b, 0, i)),
            pl.BlockSpec((Co, 2), lambda b, i: (0, 0)),
        ],
        out_shape=[
            jax.ShapeDtypeStruct((B, Co, n), jnp.float32),
            jax.ShapeDtypeStruct((Co, 2), jnp.float32),
        ],
    )(h1, st1, g1c, be1c, W2, b2c)


def _bn3_kernel(h_ref, st_ref, g_ref, be_ref, o_ref, *, count):
    st = st_ref[...]
    mean = st[:, 0:1] * (1.0 / count)
    var = st[:, 1:2] * (1.0 / count) - mean * mean
    scale = g_ref[...] * jax.lax.rsqrt(var + 1e-5)
    o_ref[0] = jnp.maximum((h_ref[0] - mean) * scale + be_ref[...], 0.0)


def _bn3(h2, st2, g2c, be2c, count):
    B, C, n = h2.shape
    NB = min(n, 1024)
    grid = (B, n // NB)
    return pl.pallas_call(
        functools.partial(_bn3_kernel, count=count),
        grid=grid,
        in_specs=[
            pl.BlockSpec((1, C, NB), lambda b, i: (b, 0, i)),
            pl.BlockSpec((C, 2), lambda b, i: (0, 0)),
            pl.BlockSpec((C, 1), lambda b, i: (0, 0)),
            pl.BlockSpec((C, 1), lambda b, i: (0, 0)),
        ],
        out_specs=pl.BlockSpec((1, C, NB), lambda b, i: (b, 0, i)),
        out_shape=jax.ShapeDtypeStruct((B, C, n), jnp.float32),
    )(h2, st2, g2c, be2c)


def kernel(unknown, known, unknow_feats, known_feats,
           W1, b1, g1, be1, W2, b2, g2, be2):
    B, n, _ = unknown.shape
    Co1 = W1.shape[0]
    Co2 = W2.shape[0]
    count = float(B * n)

    unknown_t = jnp.transpose(unknown, (0, 2, 1))
    C2 = known_feats.shape[1]
    idx3, w3 = _three_nn(unknown_t, known)
    h1b = _mlp1b(unknow_feats, W1[:, C2:], b1.reshape(Co1, 1))
    interp = _sc_interp(idx3, w3, known_feats)

    h1, st1 = _mlp1(interp, h1b, W1[:, :C2])
    h2, st2 = _mlp2(h1, st1, g1.reshape(Co1, 1), be1.reshape(Co1, 1),
                    W2, b2.reshape(Co2, 1), count)
    out = _bn3(h2, st2, g2.reshape(Co2, 1), be2.reshape(Co2, 1), count)
    return out


# speedup vs baseline: 1.4970x; 1.4970x over previous
import functools

import jax
import jax.numpy as jnp
from jax.experimental import pallas as pl


def _interp_mlp1_kernel(idx_ref, w_ref, kf_ref, hb_ref, W1a_ref,
                        h_ref, st_ref, *, m):
    idx = idx_ref[0]
    w = w_ref[0]
    iota = jax.lax.broadcasted_iota(jnp.int32, (m, idx.shape[1]), 0)
    S = None
    for k in range(3):
        sel = jnp.where(iota == idx[k:k + 1], w[k:k + 1], 0.0)
        S = sel if S is None else S + sel
    interp = jnp.dot(kf_ref[0], S, preferred_element_type=jnp.float32)
    h = hb_ref[0] + jnp.dot(W1a_ref[...], interp,
                            preferred_element_type=jnp.float32)
    h_ref[0] = h

    @pl.when(jnp.logical_and(pl.program_id(0) == 0, pl.program_id(1) == 0))
    def _():
        st_ref[...] = jnp.zeros_like(st_ref)

    s0 = jnp.sum(h, axis=1, keepdims=True)
    s1 = jnp.sum(h * h, axis=1, keepdims=True)
    st_ref[...] += jnp.concatenate([s0, s1], axis=1)


def _interp_mlp1(idx3, w3, known_feats, h1b, W1a):
    B, C2, m = known_feats.shape
    n = idx3.shape[2]
    Co = W1a.shape[0]
    NB = min(n, 512)
    grid = (B, n // NB)
    return pl.pallas_call(
        functools.partial(_interp_mlp1_kernel, m=m),
        grid=grid,
        in_specs=[
            pl.BlockSpec((1, 3, NB), lambda b, i: (b, 0, i)),
            pl.BlockSpec((1, 3, NB), lambda b, i: (b, 0, i)),
            pl.BlockSpec((1, C2, m), lambda b, i: (b, 0, 0)),
            pl.BlockSpec((1, Co, NB), lambda b, i: (b, 0, i)),
            pl.BlockSpec((Co, C2), lambda b, i: (0, 0)),
        ],
        out_specs=[
            pl.BlockSpec((1, Co, NB), lambda b, i: (b, 0, i)),
            pl.BlockSpec((Co, 2), lambda b, i: (0, 0)),
        ],
        out_shape=[
            jax.ShapeDtypeStruct((B, Co, n), jnp.float32),
            jax.ShapeDtypeStruct((Co, 2), jnp.float32),
        ],
    )(idx3, w3, known_feats, h1b, W1a)


def _nn_kernel(unk_ref, kn_ref, idx_ref, w_ref, *, m):
    u = unk_ref[0]
    kn = kn_ref[0]
    d2 = None
    for c in range(3):
        diff = kn[:, c:c + 1] - u[c].reshape(1, -1)
        sq = diff * diff
        d2 = sq if d2 is None else d2 + sq
    iota = jax.lax.broadcasted_iota(jnp.int32, d2.shape, 0)
    key = jax.lax.bitcast_convert_type(d2, jnp.int32)
    key = jax.lax.bitwise_or(jax.lax.bitwise_and(key, ~(m - 1)), iota)
    idxs, dists = [], []
    for _ in range(3):
        mn = jnp.min(key, axis=0, keepdims=True)
        key = jnp.where(key == mn, jnp.int32(0x7FFFFFFF), key)
        idxs.append(jax.lax.bitwise_and(mn, m - 1))
        d2sel = jax.lax.bitcast_convert_type(
            jax.lax.bitwise_and(mn, ~(m - 1)), jnp.float32)
        dists.append(jnp.sqrt(d2sel))
    idx_ref[0] = jnp.concatenate(idxs, axis=0)
    dr = [1.0 / (dd + 1e-8) for dd in dists]
    norm = dr[0] + dr[1] + dr[2]
    w_ref[0] = jnp.concatenate([x / norm for x in dr], axis=0)


def _three_nn(unknown_t, known):
    B, _, n = unknown_t.shape
    m = known.shape[1]
    NB = min(n, 2048)
    grid = (B, n // NB)
    return pl.pallas_call(
        functools.partial(_nn_kernel, m=m),
        grid=grid,
        in_specs=[
            pl.BlockSpec((1, 3, NB), lambda b, i: (b, 0, i)),
            pl.BlockSpec((1, m, 3), lambda b, i: (b, 0, 0)),
        ],
        out_specs=[
            pl.BlockSpec((1, 3, NB), lambda b, i: (b, 0, i)),
            pl.BlockSpec((1, 3, NB), lambda b, i: (b, 0, i)),
        ],
        out_shape=[
            jax.ShapeDtypeStruct((B, 3, n), jnp.int32),
            jax.ShapeDtypeStruct((B, 3, n), jnp.float32),
        ],
    )(unknown_t, known)


def _mlp1b_kernel(uf_ref, W1b_ref, b1_ref, o_ref):
    o_ref[0] = jnp.dot(W1b_ref[...], uf_ref[0],
                       preferred_element_type=jnp.float32) + b1_ref[...]


def _mlp1b(unknow_feats, W1b, b1c):
    B, C1, n = unknow_feats.shape
    Co = W1b.shape[0]
    NB = min(n, 1024)
    grid = (B, n // NB)
    return pl.pallas_call(
        _mlp1b_kernel,
        grid=grid,
        in_specs=[
            pl.BlockSpec((1, C1, NB), lambda b, i: (b, 0, i)),
            pl.BlockSpec((Co, C1), lambda b, i: (0, 0)),
            pl.BlockSpec((Co, 1), lambda b, i: (0, 0)),
        ],
        out_specs=pl.BlockSpec((1, Co, NB), lambda b, i: (b, 0, i)),
        out_shape=jax.ShapeDtypeStruct((B, Co, n), jnp.float32),
    )(unknow_feats, W1b, b1c)


def _mlp2_kernel(h_ref, st_ref, g_ref, be_ref, W_ref, b_ref,
                 o_ref, st2_ref, *, count):
    st = st_ref[...]
    mean = st[:, 0:1] * (1.0 / count)
    var = st[:, 1:2] * (1.0 / count) - mean * mean
    scale = g_ref[...] * jax.lax.rsqrt(var + 1e-5)
    x = jnp.maximum((h_ref[0] - mean) * scale + be_ref[...], 0.0)
    h2 = jnp.dot(W_ref[...], x, preferred_element_type=jnp.float32) + b_ref[...]
    o_ref[0] = h2

    @pl.when(jnp.logical_and(pl.program_id(0) == 0, pl.program_id(1) == 0))
    def _():
        st2_ref[...] = jnp.zeros_like(st2_ref)

    s0 = jnp.sum(h2, axis=1, keepdims=True)
    s1 = jnp.sum(h2 * h2, axis=1, keepdims=True)
    st2_ref[...] += jnp.concatenate([s0, s1], axis=1)


def _mlp2(h1, st1, g1c, be1c, W2, b2c, count):
    B, Ci, n = h1.shape
    Co = W2.shape[0]
    NB = min(n, 1024)
    grid = (B, n // NB)
    return pl.pallas_call(
        functools.partial(_mlp2_kernel, count=count),
        grid=grid,
        in_specs=[
            pl.BlockSpec((1, Ci, NB), lambda b, i: (b, 0, i)),
            pl.BlockSpec((Ci, 2), lambda b, i: (0, 0)),
            pl.BlockSpec((Ci, 1), lambda b, i: (0, 0)),
            pl.BlockSpec((Ci, 1), lambda b, i: (0, 0)),
            pl.BlockSpec((Co, Ci), lambda b, i: (0, 0)),
            pl.BlockSpec((Co, 1), lambda b, i: (0, 0)),
        ],
        out_specs=[
            pl.BlockSpec((1, Co, NB), lambda b, i: (b, 0, i)),
            pl.BlockSpec((Co, 2), lambda b, i: (0, 0)),
        ],
        out_shape=[
            jax.ShapeDtypeStruct((B, Co, n), jnp.float32),
            jax.ShapeDtypeStruct((Co, 2), jnp.float32),
        ],
    )(h1, st1, g1c, be1c, W2, b2c)


def _bn3_kernel(h_ref, st_ref, g_ref, be_ref, o_ref, *, count):
    st = st_ref[...]
    mean = st[:, 0:1] * (1.0 / count)
    var = st[:, 1:2] * (1.0 / count) - mean * mean
    scale = g_ref[...] * jax.lax.rsqrt(var + 1e-5)
    o_ref[0] = jnp.maximum((h_ref[0] - mean) * scale + be_ref[...], 0.0)


def _bn3(h2, st2, g2c, be2c, count):
    B, C, n = h2.shape
    NB = min(n, 1024)
    grid = (B, n // NB)
    return pl.pallas_call(
        functools.partial(_bn3_kernel, count=count),
        grid=grid,
        in_specs=[
            pl.BlockSpec((1, C, NB), lambda b, i: (b, 0, i)),
            pl.BlockSpec((C, 2), lambda b, i: (0, 0)),
            pl.BlockSpec((C, 1), lambda b, i: (0, 0)),
            pl.BlockSpec((C, 1), lambda b, i: (0, 0)),
        ],
        out_specs=pl.BlockSpec((1, C, NB), lambda b, i: (b, 0, i)),
        out_shape=jax.ShapeDtypeStruct((B, C, n), jnp.float32),
    )(h2, st2, g2c, be2c)


def kernel(unknown, known, unknow_feats, known_feats,
           W1, b1, g1, be1, W2, b2, g2, be2):
    B, n, _ = unknown.shape
    Co1 = W1.shape[0]
    Co2 = W2.shape[0]
    count = float(B * n)

    unknown_t = jnp.transpose(unknown, (0, 2, 1))
    C2 = known_feats.shape[1]
    idx3, w3 = _three_nn(unknown_t, known)
    h1b = _mlp1b(unknow_feats, W1[:, C2:], b1.reshape(Co1, 1))
    h1, st1 = _interp_mlp1(idx3, w3, known_feats, h1b, W1[:, :C2])
    h2, st2 = _mlp2(h1, st1, g1.reshape(Co1, 1), be1.reshape(Co1, 1),
                    W2, b2.reshape(Co2, 1), count)
    out = _bn3(h2, st2, g2.reshape(Co2, 1), be2.reshape(Co2, 1), count)
    return out
